# Initial kernel scaffold; baseline (speedup 1.0000x reference)
import jax
import jax.numpy as jnp
from jax.experimental import pallas as pl


def kernel(x, W1, b1, W2, b2, gamma, beta):
    raise NotImplementedError("write your pallas kernel here")


# speedup vs baseline: 2.6021x; 2.6021x over previous
import jax
import jax.numpy as jnp
from jax import lax
from jax.experimental import pallas as pl
from jax.experimental.pallas import tpu as pltpu

B, S, D = 4, 8192, 768
F = D // 4
CAP = S // 2
TS = 1024
NTOK = B * S
NGRID = NTOK // TS


def _sortkey(f32vec):
    i = lax.bitcast_convert_type(f32vec, jnp.int32)
    return i ^ ((i >> 31) & jnp.int32(0x7FFFFFFF))


def _router_body(x_ref, w1_ref, b1_ref, w2_ref, b2_ref, s_ref, rw_ref):
    xb = x_ref[...]
    h = jnp.dot(xb, w1_ref[...], preferred_element_type=jnp.float32)
    h = jnp.maximum(h + b1_ref[0][None, :], 0.0)
    s = jnp.sum(h * w2_ref[...], axis=1) + b2_ref[0, 0]
    s_ref[...] = s
    rw_ref[...] = jax.nn.sigmoid(s)


def _thresh_body(s_ref, v_ref):
    keys = _sortkey(s_ref[0, 0])

    def body(_, carry):
        lo, hi = carry
        mid = (lo >> 1) + (hi >> 1) + (lo & hi & 1) + ((lo ^ hi) & 1)
        cnt = jnp.sum((keys >= mid).astype(jnp.int32))
        pred = cnt >= CAP
        lo2 = jnp.where(pred, mid, lo)
        hi2 = jnp.where(pred, hi, mid - 1)
        take = lo < hi
        return (jnp.where(take, lo2, lo), jnp.where(take, hi2, hi))

    lo, _ = lax.fori_loop(
        0, 33, body, (jnp.int32(-(2**31)), jnp.int32(2**31 - 1)))
    v_ref[0, 0] = jnp.full((128,), lo, jnp.int32)


def _apply_body(vk_ref, x_ref, s_ref, g_ref, bt_ref, o_ref):
    b = pl.program_id(0) // (S // TS)
    xb = x_ref[...]
    keys = _sortkey(s_ref[...])
    mask = keys >= vk_ref[b]
    o_ref[...] = jnp.where(
        mask, xb * g_ref[0][None, :] + bt_ref[0][None, :], xb)


@jax.jit
def kernel(x, W1, b1, W2, b2, gamma, beta):
    x2 = x.reshape(NTOK, D)

    scores, rw = pl.pallas_call(
        _router_body,
        grid=(NGRID,),
        in_specs=[
            pl.BlockSpec((TS, D), lambda t: (t, 0)),
            pl.BlockSpec((D, F), lambda t: (0, 0)),
            pl.BlockSpec((1, F), lambda t: (0, 0)),
            pl.BlockSpec((1, F), lambda t: (0, 0)),
            pl.BlockSpec((1, 1), lambda t: (0, 0), memory_space=pltpu.SMEM),
        ],
        out_specs=[
            pl.BlockSpec((TS,), lambda t: (t,)),
            pl.BlockSpec((TS,), lambda t: (t,)),
        ],
        out_shape=[
            jax.ShapeDtypeStruct((NTOK,), jnp.float32),
            jax.ShapeDtypeStruct((NTOK,), jnp.float32),
        ],
    )(x2, W1, b1.reshape(1, F), W2.reshape(1, F), b2.reshape(1, 1))

    vkeys3 = pl.pallas_call(
        _thresh_body,
        grid=(B,),
        in_specs=[pl.BlockSpec((1, 1, S), lambda b: (b, 0, 0))],
        out_specs=pl.BlockSpec((1, 1, 128), lambda b: (b, 0, 0)),
        out_shape=jax.ShapeDtypeStruct((B, 1, 128), jnp.int32),
    )(scores.reshape(B, 1, S))
    vkeys = vkeys3[:, 0, 0]

    out = pl.pallas_call(
        _apply_body,
        grid=(NGRID,),
        in_specs=[
            pl.BlockSpec(memory_space=pltpu.SMEM),
            pl.BlockSpec((TS, D), lambda t: (t, 0)),
            pl.BlockSpec((TS, 1), lambda t: (t, 0)),
            pl.BlockSpec((1, D), lambda t: (0, 0)),
            pl.BlockSpec((1, D), lambda t: (0, 0)),
        ],
        out_specs=pl.BlockSpec((TS, D), lambda t: (t, 0)),
        out_shape=jax.ShapeDtypeStruct((NTOK, D), jnp.float32),
    )(vkeys, x2, scores.reshape(NTOK, 1), gamma.reshape(1, D),
      beta.reshape(1, D))

    return out.reshape(B, S, D), rw.reshape(B, S, 1)


# speedup vs baseline: 2.9894x; 1.1489x over previous
import jax
import jax.numpy as jnp
from jax import lax
from jax.experimental import pallas as pl
from jax.experimental.pallas import tpu as pltpu

B, S, D = 4, 8192, 768
F = D // 4
CAP = S // 2
TS = 1024
NTOK = B * S
NGRID = NTOK // TS


def _sortkey(f32vec):
    i = lax.bitcast_convert_type(f32vec, jnp.int32)
    return i ^ ((i >> 31) & jnp.int32(0x7FFFFFFF))


def _router_body(x_ref, w1_ref, b1_ref, w2p_ref, b2_ref, s_ref, rw_ref):
    xb = x_ref[...]
    h = jnp.dot(xb, w1_ref[...], preferred_element_type=jnp.float32)
    h = jnp.maximum(h + b1_ref[0][None, :], 0.0)
    sm = jnp.dot(h, w2p_ref[...], preferred_element_type=jnp.float32)
    s = sm[:, 0:1] + b2_ref[0, 0]
    s_ref[...] = s
    rw_ref[...] = jax.nn.sigmoid(s)


def _thresh_body(s_ref, v_ref):
    keys = _sortkey(s_ref[...])

    def body(_, carry):
        lo, hi = carry
        mid = (lo >> 1) + (hi >> 1) + (lo & hi & 1) + ((lo ^ hi) & 1)
        cnt = jnp.sum((keys >= mid).astype(jnp.int32), axis=1, keepdims=True)
        pred = cnt >= CAP
        lo2 = jnp.where(pred, mid, lo)
        hi2 = jnp.where(pred, hi, mid - 1)
        take = lo < hi
        return (jnp.where(take, lo2, lo), jnp.where(take, hi2, hi))

    init = (jnp.full((B, 1), -(2**31), jnp.int32),
            jnp.full((B, 1), 2**31 - 1, jnp.int32))
    lo, _ = lax.fori_loop(0, 33, body, init)
    v_ref[...] = jnp.broadcast_to(lo, (B, 128))


def _apply_body(vk_ref, x_ref, s_ref, g_ref, bt_ref, o_ref):
    b = pl.program_id(0) // (S // TS)
    xb = x_ref[...]
    keys = _sortkey(s_ref[...])
    mask = keys >= vk_ref[b]
    o_ref[...] = jnp.where(
        mask, xb * g_ref[0][None, :] + bt_ref[0][None, :], xb)


@jax.jit
def kernel(x, W1, b1, W2, b2, gamma, beta):
    x2 = x.reshape(NTOK, D)
    w2p = jnp.pad(W2, ((0, 0), (0, 127)))

    scores, rw = pl.pallas_call(
        _router_body,
        grid=(NGRID,),
        in_specs=[
            pl.BlockSpec((TS, D), lambda t: (t, 0)),
            pl.BlockSpec((D, F), lambda t: (0, 0)),
            pl.BlockSpec((1, F), lambda t: (0, 0)),
            pl.BlockSpec((F, 128), lambda t: (0, 0)),
            pl.BlockSpec((1, 1), lambda t: (0, 0), memory_space=pltpu.SMEM),
        ],
        out_specs=[
            pl.BlockSpec((TS, 1), lambda t: (t, 0)),
            pl.BlockSpec((TS, 1), lambda t: (t, 0)),
        ],
        out_shape=[
            jax.ShapeDtypeStruct((NTOK, 1), jnp.float32),
            jax.ShapeDtypeStruct((NTOK, 1), jnp.float32),
        ],
    )(x2, W1, b1.reshape(1, F), w2p, b2.reshape(1, 1))

    vkeys2 = pl.pallas_call(
        _thresh_body,
        in_specs=[pl.BlockSpec((B, S), lambda: (0, 0))],
        out_specs=pl.BlockSpec((B, 128), lambda: (0, 0)),
        out_shape=jax.ShapeDtypeStruct((B, 128), jnp.int32),
    )(scores.reshape(B, S))
    vkeys = vkeys2[:, 0]

    out = pl.pallas_call(
        _apply_body,
        grid=(NGRID,),
        in_specs=[
            pl.BlockSpec(memory_space=pltpu.SMEM),
            pl.BlockSpec((TS, D), lambda t: (t, 0)),
            pl.BlockSpec((TS, 1), lambda t: (t, 0)),
            pl.BlockSpec((1, D), lambda t: (0, 0)),
            pl.BlockSpec((1, D), lambda t: (0, 0)),
        ],
        out_specs=pl.BlockSpec((TS, D), lambda t: (t, 0)),
        out_shape=jax.ShapeDtypeStruct((NTOK, D), jnp.float32),
    )(vkeys, x2, scores, gamma.reshape(1, D), beta.reshape(1, D))

    return out.reshape(B, S, D), rw.reshape(B, S, 1)


# speedup vs baseline: 3.0324x; 1.0144x over previous
import jax
import jax.numpy as jnp
from jax import lax
from jax.experimental import pallas as pl
from jax.experimental.pallas import tpu as pltpu

B, S, D = 4, 8192, 768
F = D // 4
CAP = S // 2
TS = 1024
NTOK = B * S
NGRID = NTOK // TS


def _sortkey(f32vec):
    i = lax.bitcast_convert_type(f32vec, jnp.int32)
    return i ^ ((i >> 31) & jnp.int32(0x7FFFFFFF))


def _router_body(x_ref, w1_ref, b1_ref, w2p_ref, b2_ref, s_ref, rw_ref):
    xb = x_ref[...]
    h = jnp.dot(xb, w1_ref[...], preferred_element_type=jnp.float32)
    h = jnp.maximum(h + b1_ref[0][None, :], 0.0)
    sm = jnp.dot(h, w2p_ref[...], preferred_element_type=jnp.float32)
    s = sm[:, 0:1] + b2_ref[0, 0]
    s_ref[...] = s
    rw_ref[...] = jax.nn.sigmoid(s)


def _thresh_apply_body(sb_ref, x_ref, s_ref, g_ref, bt_ref, o_ref, vk_ref):
    t = pl.program_id(0)

    @pl.when(t == 0)
    def _thresh():
        keys = _sortkey(sb_ref[...])

        def body(_, carry):
            lo, hi = carry
            mid = (lo >> 1) + (hi >> 1) + (lo & hi & 1) + ((lo ^ hi) & 1)
            cnt = jnp.sum((keys >= mid).astype(jnp.int32),
                          axis=1, keepdims=True)
            pred = cnt >= CAP
            lo2 = jnp.where(pred, mid, lo)
            hi2 = jnp.where(pred, hi, mid - 1)
            take = lo < hi
            return (jnp.where(take, lo2, lo), jnp.where(take, hi2, hi))

        init = (jnp.full((B, 1), -(2**31), jnp.int32),
                jnp.full((B, 1), 2**31 - 1, jnp.int32))
        lo, _ = lax.fori_loop(0, 33, body, init)
        for b in range(B):
            vk_ref[b] = lo[b, 0]

    @pl.when(t > 0)
    def _apply():
        b = (t - 1) // (S // TS)
        xb = x_ref[...]
        keys = _sortkey(s_ref[...])
        mask = keys >= vk_ref[b]
        o_ref[...] = jnp.where(
            mask, xb * g_ref[0][None, :] + bt_ref[0][None, :], xb)


@jax.jit
def kernel(x, W1, b1, W2, b2, gamma, beta):
    x2 = x.reshape(NTOK, D)
    w2p = jnp.pad(W2, ((0, 0), (0, 127)))

    scores, rw = pl.pallas_call(
        _router_body,
        grid=(NGRID,),
        in_specs=[
            pl.BlockSpec((TS, D), lambda t: (t, 0)),
            pl.BlockSpec((D, F), lambda t: (0, 0)),
            pl.BlockSpec((1, F), lambda t: (0, 0)),
            pl.BlockSpec((F, 128), lambda t: (0, 0)),
            pl.BlockSpec((1, 1), lambda t: (0, 0), memory_space=pltpu.SMEM),
        ],
        out_specs=[
            pl.BlockSpec((TS, 1), lambda t: (t, 0)),
            pl.BlockSpec((TS, 1), lambda t: (t, 0)),
        ],
        out_shape=[
            jax.ShapeDtypeStruct((NTOK, 1), jnp.float32),
            jax.ShapeDtypeStruct((NTOK, 1), jnp.float32),
        ],
    )(x2, W1, b1.reshape(1, F), w2p, b2.reshape(1, 1))

    out = pl.pallas_call(
        _thresh_apply_body,
        grid=(NGRID + 1,),
        in_specs=[
            pl.BlockSpec((B, S), lambda t: (0, 0)),
            pl.BlockSpec((TS, D), lambda t: (jnp.maximum(t - 1, 0), 0)),
            pl.BlockSpec((TS, 1), lambda t: (jnp.maximum(t - 1, 0), 0)),
            pl.BlockSpec((1, D), lambda t: (0, 0)),
            pl.BlockSpec((1, D), lambda t: (0, 0)),
        ],
        out_specs=pl.BlockSpec((TS, D), lambda t: (jnp.maximum(t - 1, 0), 0)),
        out_shape=jax.ShapeDtypeStruct((NTOK, D), jnp.float32),
        scratch_shapes=[pltpu.SMEM((B,), jnp.int32)],
    )(scores.reshape(B, S), x2, scores, gamma.reshape(1, D),
      beta.reshape(1, D))

    return out.reshape(B, S, D), rw.reshape(B, S, 1)


# speedup vs baseline: 3.2919x; 1.0856x over previous
import jax
import jax.numpy as jnp
from jax import lax
from jax.experimental import pallas as pl
from jax.experimental.pallas import tpu as pltpu

B, S, D = 4, 8192, 768
F = D // 4
CAP = S // 2
TSR = 2048
TSA = 2048
NTOK = B * S
NGR = NTOK // TSR
NGA = NTOK // TSA


def _sortkey(f32vec):
    i = lax.bitcast_convert_type(f32vec, jnp.int32)
    return i ^ ((i >> 31) & jnp.int32(0x7FFFFFFF))


def _router_body(x_ref, w1_ref, b1_ref, w2p_ref, b2_ref, s_ref, rw_ref):
    xb = x_ref[...]
    h = jnp.dot(xb, w1_ref[...], preferred_element_type=jnp.float32)
    h = jnp.maximum(h + b1_ref[0][None, :], 0.0)
    sm = jnp.dot(h, w2p_ref[...], preferred_element_type=jnp.float32)
    s = sm[:, 0:1] + b2_ref[0, 0]
    s_ref[...] = s
    rw_ref[...] = jax.nn.sigmoid(s)


def _thresh_apply_body(sb_ref, x_ref, s_ref, g_ref, bt_ref, o_ref, vk_ref):
    t = pl.program_id(0)

    @pl.when(t == 0)
    def _thresh():
        keys = _sortkey(sb_ref[...])

        def body(_, carry):
            lo, hi = carry
            mid = (lo >> 1) + (hi >> 1) + (lo & hi & 1) + ((lo ^ hi) & 1)
            cnt = jnp.sum((keys >= mid).astype(jnp.int32),
                          axis=1, keepdims=True)
            pred = cnt >= CAP
            lo2 = jnp.where(pred, mid, lo)
            hi2 = jnp.where(pred, hi, mid - 1)
            take = lo < hi
            return (jnp.where(take, lo2, lo), jnp.where(take, hi2, hi))

        init = (jnp.full((B, 1), -(2**31), jnp.int32),
                jnp.full((B, 1), 2**31 - 1, jnp.int32))
        lo, _ = lax.fori_loop(0, 33, body, init)
        for b in range(B):
            vk_ref[b] = lo[b, 0]

    @pl.when(t > 0)
    def _apply():
        b = (t - 1) // (S // TSA)
        xb = x_ref[...]
        keys = _sortkey(s_ref[...])
        mask = keys >= vk_ref[b]
        o_ref[...] = jnp.where(
            mask, xb * g_ref[0][None, :] + bt_ref[0][None, :], xb)


@jax.jit
def kernel(x, W1, b1, W2, b2, gamma, beta):
    x2 = x.reshape(NTOK, D)
    w2p = jnp.pad(W2, ((0, 0), (0, 127)))

    scores, rw = pl.pallas_call(
        _router_body,
        grid=(NGR,),
        in_specs=[
            pl.BlockSpec((TSR, D), lambda t: (t, 0)),
            pl.BlockSpec((D, F), lambda t: (0, 0)),
            pl.BlockSpec((1, F), lambda t: (0, 0)),
            pl.BlockSpec((F, 128), lambda t: (0, 0)),
            pl.BlockSpec((1, 1), lambda t: (0, 0), memory_space=pltpu.SMEM),
        ],
        out_specs=[
            pl.BlockSpec((TSR, 1), lambda t: (t, 0)),
            pl.BlockSpec((TSR, 1), lambda t: (t, 0)),
        ],
        out_shape=[
            jax.ShapeDtypeStruct((NTOK, 1), jnp.float32),
            jax.ShapeDtypeStruct((NTOK, 1), jnp.float32),
        ],
    )(x2, W1, b1.reshape(1, F), w2p, b2.reshape(1, 1))

    out = pl.pallas_call(
        _thresh_apply_body,
        grid=(NGA + 1,),
        in_specs=[
            pl.BlockSpec((B, S), lambda t: (0, 0)),
            pl.BlockSpec((TSA, D), lambda t: (jnp.maximum(t - 1, 0), 0)),
            pl.BlockSpec((TSA, 1), lambda t: (jnp.maximum(t - 1, 0), 0)),
            pl.BlockSpec((1, D), lambda t: (0, 0)),
            pl.BlockSpec((1, D), lambda t: (0, 0)),
        ],
        out_specs=pl.BlockSpec((TSA, D), lambda t: (jnp.maximum(t - 1, 0), 0)),
        out_shape=jax.ShapeDtypeStruct((NTOK, D), jnp.float32),
        scratch_shapes=[pltpu.SMEM((B,), jnp.int32)],
    )(scores.reshape(B, S), x2, scores, gamma.reshape(1, D),
      beta.reshape(1, D))

    return out.reshape(B, S, D), rw.reshape(B, S, 1)
